# baseline (speedup 1.0000x reference)
import jax
import jax.numpy as jnp
import numpy as np
from jax.experimental import pallas as pl

EMBED_SIZE = 768
D_TYPE = EMBED_SIZE // 4
MAX_SEQ = 24
BASE_GSD = 10.0


def _sincos_1d(pos, dim):
    omega = 1.0 / (10000.0 ** (jnp.arange(dim // 2, dtype=jnp.float32) / (dim / 2.0)))
    out = pos.astype(jnp.float32)[:, None] * omega[None, :]
    return jnp.concatenate([jnp.sin(out), jnp.cos(out)], axis=-1)


def _month_table(dim):
    angles = jnp.arange(0, 13, dtype=jnp.float32) / (12.0 / (2.0 * np.pi))
    ang = jnp.stack([angles] * (dim // 2), axis=-1)
    return jnp.concatenate([jnp.sin(ang)[:-1], jnp.cos(ang)[:-1]], axis=-1)


def _emb_from_grid_1d(pos, dim):
    omega = 1.0 / (10000.0 ** (jnp.arange(dim // 2, dtype=jnp.float32) / (dim / 2.0)))
    flat = pos.reshape(pos.shape[0], -1)
    out = flat[..., None] * omega[None, None, :]
    return jnp.concatenate([jnp.sin(out), jnp.cos(out)], axis=-1)


def _spatial_table(grid_size, res, dim):
    coords = jnp.arange(grid_size, dtype=jnp.float32)
    gw, gh = jnp.meshgrid(coords, coords, indexing='xy')
    grid = jnp.stack([gw, gh], axis=0)
    grid = grid[None, :, :, :] * res[:, None, None, None]
    emb_h = _emb_from_grid_1d(grid[:, 0], dim // 2)
    emb_w = _emb_from_grid_1d(grid[:, 1], dim // 2)
    return jnp.concatenate([emb_h, emb_w], axis=-1)


def _add_kernel(months_ref, tok_ref, a_ref, s_ref, mt_ref, out_ref):
    tok = tok_ref[0, 0]
    a = a_ref[...]
    s = s_ref[0, 0]
    m_ids = months_ref[0, 0]
    t = m_ids.shape[0]
    iota = jax.lax.broadcasted_iota(jnp.int32, (t, 12), 1)
    onehot = (m_ids[:, None] == iota).astype(jnp.float32)
    mo = jnp.dot(onehot, mt_ref[...], preferred_element_type=jnp.float32)
    out_ref[0, 0] = (tok + a[None, :, :, :]
                     + mo[None, :, None, :]
                     + s[:, None, None, :])


@jax.jit
def _run(tokens, a_table, s_table, months3, mtable):
    b, h, w, t, c, d = tokens.shape
    grid = (b, h)
    return pl.pallas_call(
        _add_kernel,
        grid=grid,
        in_specs=[
            pl.BlockSpec((1, 1, t), lambda i, j: (i, 0, 0)),
            pl.BlockSpec((1, 1, w, t, c, d), lambda i, j: (i, j, 0, 0, 0, 0)),
            pl.BlockSpec((t, c, d), lambda i, j: (0, 0, 0)),
            pl.BlockSpec((1, 1, w, d), lambda i, j: (i, j, 0, 0)),
            pl.BlockSpec((12, d), lambda i, j: (0, 0)),
        ],
        out_specs=pl.BlockSpec((1, 1, w, t, c, d), lambda i, j: (i, j, 0, 0, 0, 0)),
        out_shape=jax.ShapeDtypeStruct(tokens.shape, tokens.dtype),
    )(months3, tokens, a_table, s_table, mtable)


def kernel(tokens, channel_embeddings, timestamps, patch_size, input_res):
    b, h, w, t, c, d = tokens.shape
    dt = d // 4

    pos = _sincos_1d(jnp.arange(MAX_SEQ), dt)[:t]
    zeros_tc = jnp.zeros((t, c, 2 * dt), dtype=jnp.float32)
    a_table = jnp.concatenate(
        [jnp.broadcast_to(channel_embeddings[None, :, :], (t, c, dt)),
         jnp.broadcast_to(pos[:, None, :], (t, c, dt)),
         zeros_tc], axis=-1)

    gsd_ratio = (jnp.asarray(input_res).astype(jnp.float32)
                 * jnp.asarray(patch_size).astype(jnp.float32) / BASE_GSD)
    spatial = _spatial_table(h, jnp.ones((b,), dtype=jnp.float32) * gsd_ratio, dt)
    spatial = spatial.reshape(b, h, w, dt)
    s_table = jnp.concatenate(
        [jnp.zeros((b, h, w, 3 * dt), dtype=jnp.float32), spatial], axis=-1)

    mtable = jnp.concatenate(
        [jnp.zeros((12, 2 * dt), dtype=jnp.float32), _month_table(dt),
         jnp.zeros((12, dt), dtype=jnp.float32)], axis=-1)

    months3 = timestamps[:, 1, :].astype(jnp.int32).reshape(b, 1, t)

    return _run(tokens, a_table, s_table, months3, mtable)
